# Initial kernel scaffold; baseline (speedup 1.0000x reference)
import jax
import jax.numpy as jnp
from jax.experimental import pallas as pl


def kernel(src, dst, ntype_dict, embeddings, W, b):
    raise NotImplementedError("write your pallas kernel here")


# speedup vs baseline: 113.1168x; 113.1168x over previous
import functools

import jax
import jax.numpy as jnp
from jax import lax
from jax.experimental import pallas as pl
from jax.experimental.pallas import tpu as pltpu
from jax.experimental.pallas import tpu_sc as plsc

N = 10000
E = 320000
D = 128
T = 8

NC = 2
NS = 16
NW = NC * NS
EDGES_PER_W = E // NW
EVENTS_PER_W = 2 * EDGES_PER_W
HIST = N * T
HIST_SLICE = HIST // NS


def _sc_hist_kernel(src_hbm, dst_hbm, ntype_hbm, zeros_hbm, ones_hbm,
                    out_hbm, ntype_v, src_v, dst_v, idx_v, ones_v, zeros_v,
                    counts_sh):
    cid = lax.axis_index("c")
    sid = lax.axis_index("s")
    wid = sid * NC + cid
    base = wid * EDGES_PER_W

    pltpu.sync_copy(ntype_hbm, ntype_v)
    pltpu.sync_copy(src_hbm.at[pl.ds(base, EDGES_PER_W)], src_v)
    pltpu.sync_copy(dst_hbm.at[pl.ds(base, EDGES_PER_W)], dst_v)
    pltpu.sync_copy(ones_hbm, ones_v)
    pltpu.sync_copy(zeros_hbm.at[pl.ds(sid * HIST_SLICE, HIST_SLICE)], zeros_v)
    pltpu.sync_copy(zeros_v, counts_sh.at[pl.ds(sid * HIST_SLICE, HIST_SLICE)])

    def body(i, _):
        off = i * 16
        s = src_v[pl.ds(off, 16)]
        d = dst_v[pl.ds(off, 16)]
        ts = plsc.load_gather(ntype_v, [s])
        td = plsc.load_gather(ntype_v, [d])
        idx_v[pl.ds(off, 16)] = d * 8 + ts
        idx_v[pl.ds(EDGES_PER_W + off, 16)] = s * 8 + td
        return 0

    lax.fori_loop(0, EDGES_PER_W // 16, body, 0)

    plsc.subcore_barrier()
    pltpu.sync_copy(ones_v, counts_sh.at[idx_v], add=True)
    plsc.subcore_barrier()

    pltpu.sync_copy(counts_sh.at[pl.ds(sid * HIST_SLICE, HIST_SLICE)], zeros_v)
    pltpu.sync_copy(zeros_v,
                    out_hbm.at[pl.ds(cid * HIST + sid * HIST_SLICE,
                                     HIST_SLICE)])


@jax.jit
def _sc_hist(src, dst, ntype_dict, zeros, ones):
    mesh = plsc.VectorSubcoreMesh(core_axis_name="c", subcore_axis_name="s")
    k = functools.partial(
        pl.kernel,
        mesh=mesh,
        out_type=jax.ShapeDtypeStruct((NC * HIST,), jnp.float32),
        scratch_types=[
            pltpu.VMEM((N,), jnp.int32),
            pltpu.VMEM((EDGES_PER_W,), jnp.int32),
            pltpu.VMEM((EDGES_PER_W,), jnp.int32),
            pltpu.VMEM((EVENTS_PER_W,), jnp.int32),
            pltpu.VMEM((EVENTS_PER_W,), jnp.float32),
            pltpu.VMEM((HIST_SLICE,), jnp.float32),
            pltpu.VMEM_SHARED((HIST,), jnp.float32),
        ],
        compiler_params=pltpu.CompilerParams(needs_layout_passes=False),
    )(_sc_hist_kernel)
    return k(src, dst, ntype_dict, zeros, ones)


def _epilogue_kernel(parts_ref, emb_ref, w_ref, b_ref, out_ref):
    c = parts_ref[0] + parts_ref[1]
    deg = jnp.maximum(jnp.sum(c, axis=1, keepdims=True), 1.0)
    m = jnp.dot(emb_ref[...], w_ref[...].T,
                preferred_element_type=jnp.float32)
    out_ref[...] = (jnp.dot(c, m, preferred_element_type=jnp.float32) / deg
                    + b_ref[...])


@jax.jit
def _epilogue(parts, embeddings, W, b):
    return pl.pallas_call(
        _epilogue_kernel,
        out_shape=jax.ShapeDtypeStruct((N, T), jnp.float32),
    )(parts, embeddings, W, b.reshape(1, T))


def kernel(src, dst, ntype_dict, embeddings, W, b):
    src = src.astype(jnp.int32)
    dst = dst.astype(jnp.int32)
    zeros = jnp.zeros((HIST,), jnp.float32)
    ones = jnp.ones((EVENTS_PER_W,), jnp.float32)
    parts = _sc_hist(src, dst, ntype_dict.astype(jnp.int32), zeros, ones)
    return _epilogue(parts.reshape(NC, N, T), embeddings, W, b)


# speedup vs baseline: 117.2459x; 1.0365x over previous
import functools

import jax
import jax.numpy as jnp
from jax import lax
from jax.experimental import pallas as pl
from jax.experimental.pallas import tpu as pltpu
from jax.experimental.pallas import tpu_sc as plsc

N = 10000
E = 320000
D = 128
T = 8

NC = 2
NS = 16
NW = NC * NS
EDGES_PER_W = E // NW
EVENTS_PER_W = 2 * EDGES_PER_W
HIST = N * T
HIST_SLICE = HIST // NS
ZBUF = HIST_SLICE + 8


def _sc_hist_kernel(src_hbm, dst_hbm, ntype_hbm, out_hbm,
                    ntype_v, src_v, dst_v, idx_v, ones_v, zeros_v,
                    counts_sh, sem):
    cid = lax.axis_index("c")
    sid = lax.axis_index("s")
    wid = sid * NC + cid
    base = wid * EDGES_PER_W

    cp_nt = pltpu.async_copy(ntype_hbm, ntype_v, sem)
    cp_src = pltpu.async_copy(src_hbm.at[pl.ds(base, EDGES_PER_W)], src_v, sem)
    cp_dst = pltpu.async_copy(dst_hbm.at[pl.ds(base, EDGES_PER_W)], dst_v, sem)

    one16 = jnp.full((16,), 1.0, jnp.float32)
    zero16 = jnp.zeros((16,), jnp.float32)

    def fill_ones(i, _):
        ones_v[pl.ds(i * 16, 16)] = one16
        return 0

    lax.fori_loop(0, EVENTS_PER_W // 16, fill_ones, 0, unroll=8)

    def fill_zeros(i, _):
        zeros_v[pl.ds(i * 16, 16)] = zero16
        return 0

    lax.fori_loop(0, ZBUF // 16, fill_zeros, 0, unroll=8)

    pltpu.sync_copy(zeros_v.at[pl.ds(0, HIST_SLICE)],
                    counts_sh.at[pl.ds(sid * HIST_SLICE, HIST_SLICE)])

    cp_nt.wait()
    cp_src.wait()
    cp_dst.wait()

    def body(i, _):
        off = i * 16
        s = src_v[pl.ds(off, 16)]
        d = dst_v[pl.ds(off, 16)]
        ts = plsc.load_gather(ntype_v, [s])
        td = plsc.load_gather(ntype_v, [d])
        idx_v[pl.ds(off, 16)] = d * 8 + ts
        idx_v[pl.ds(EDGES_PER_W + off, 16)] = s * 8 + td
        return 0

    lax.fori_loop(0, EDGES_PER_W // 16, body, 0, unroll=8)

    plsc.subcore_barrier()
    pltpu.sync_copy(ones_v, counts_sh.at[idx_v], add=True)
    plsc.subcore_barrier()

    pltpu.sync_copy(counts_sh.at[pl.ds(sid * HIST_SLICE, HIST_SLICE)],
                    zeros_v.at[pl.ds(0, HIST_SLICE)])
    pltpu.sync_copy(zeros_v.at[pl.ds(0, HIST_SLICE)],
                    out_hbm.at[pl.ds(cid * HIST + sid * HIST_SLICE,
                                     HIST_SLICE)])


@jax.jit
def _sc_hist(src, dst, ntype_dict):
    mesh = plsc.VectorSubcoreMesh(core_axis_name="c", subcore_axis_name="s")
    k = functools.partial(
        pl.kernel,
        mesh=mesh,
        out_type=jax.ShapeDtypeStruct((NC * HIST,), jnp.float32),
        scratch_types=[
            pltpu.VMEM((N,), jnp.int32),
            pltpu.VMEM((EDGES_PER_W,), jnp.int32),
            pltpu.VMEM((EDGES_PER_W,), jnp.int32),
            pltpu.VMEM((EVENTS_PER_W,), jnp.int32),
            pltpu.VMEM((EVENTS_PER_W,), jnp.float32),
            pltpu.VMEM((ZBUF,), jnp.float32),
            pltpu.VMEM_SHARED((HIST,), jnp.float32),
            pltpu.SemaphoreType.DMA,
        ],
        compiler_params=pltpu.CompilerParams(needs_layout_passes=False),
    )(_sc_hist_kernel)
    return k(src, dst, ntype_dict)


def _epilogue_kernel(parts_ref, emb_ref, w_ref, b_ref, out_ref):
    c = parts_ref[0] + parts_ref[1]
    deg = jnp.maximum(jnp.sum(c, axis=1, keepdims=True), 1.0)
    m = jnp.dot(emb_ref[...], w_ref[...].T,
                preferred_element_type=jnp.float32)
    out_ref[...] = (jnp.dot(c, m, preferred_element_type=jnp.float32) / deg
                    + b_ref[...])


@jax.jit
def _epilogue(parts, embeddings, W, b):
    return pl.pallas_call(
        _epilogue_kernel,
        out_shape=jax.ShapeDtypeStruct((N, T), jnp.float32),
    )(parts, embeddings, W, b.reshape(1, T))


def kernel(src, dst, ntype_dict, embeddings, W, b):
    src = src.astype(jnp.int32)
    dst = dst.astype(jnp.int32)
    parts = _sc_hist(src, dst, ntype_dict.astype(jnp.int32))
    return _epilogue(parts.reshape(NC, N, T), embeddings, W, b)


# speedup vs baseline: 146.4557x; 1.2491x over previous
import functools

import jax
import jax.numpy as jnp
from jax import lax
from jax.experimental import pallas as pl
from jax.experimental.pallas import tpu as pltpu
from jax.experimental.pallas import tpu_sc as plsc

N = 10000
E = 320000
D = 128
T = 8

NC = 2
NS = 16
NW = NC * NS
EDGES_PER_W = E // NW
EVENTS_PER_W = 2 * EDGES_PER_W
HIST = N * T
HIST_PAD = 81920
ROWS = HIST // 128
HIST_SLICE = HIST // NS
ZBUF = HIST_SLICE + 8


def _sc_hist_kernel(src_hbm, dst_hbm, ntype_hbm, out_hbm,
                    ntype_v, src_v, dst_v, idx_v, ones_v, zeros_v,
                    counts_sh, sem):
    cid = lax.axis_index("c")
    sid = lax.axis_index("s")
    wid = sid * NC + cid
    base = wid * EDGES_PER_W

    cp_nt = pltpu.async_copy(ntype_hbm, ntype_v, sem)
    cp_src = pltpu.async_copy(src_hbm.at[pl.ds(base, EDGES_PER_W)], src_v, sem)
    cp_dst = pltpu.async_copy(dst_hbm.at[pl.ds(base, EDGES_PER_W)], dst_v, sem)

    one16 = jnp.full((16,), 1.0, jnp.float32)
    zero16 = jnp.zeros((16,), jnp.float32)

    def fill_ones(i, _):
        ones_v[pl.ds(i * 16, 16)] = one16
        return 0

    lax.fori_loop(0, EVENTS_PER_W // 16, fill_ones, 0, unroll=8)

    def fill_zeros(i, _):
        zeros_v[pl.ds(i * 16, 16)] = zero16
        return 0

    lax.fori_loop(0, ZBUF // 16, fill_zeros, 0, unroll=8)

    pltpu.sync_copy(zeros_v.at[pl.ds(0, HIST_SLICE)],
                    counts_sh.at[pl.ds(sid * HIST_SLICE, HIST_SLICE)])
    @pl.when(sid == 0)
    def _():
        pltpu.sync_copy(zeros_v.at[pl.ds(0, HIST_PAD - HIST)],
                        out_hbm.at[pl.ds(cid * HIST_PAD + HIST,
                                         HIST_PAD - HIST)])

    cp_nt.wait()
    cp_src.wait()
    cp_dst.wait()

    def body(i, _):
        off = i * 16
        s = src_v[pl.ds(off, 16)]
        d = dst_v[pl.ds(off, 16)]
        ts = plsc.load_gather(ntype_v, [s])
        td = plsc.load_gather(ntype_v, [d])
        idx_v[pl.ds(off, 16)] = d * 8 + ts
        idx_v[pl.ds(EDGES_PER_W + off, 16)] = s * 8 + td
        return 0

    lax.fori_loop(0, EDGES_PER_W // 16, body, 0, unroll=8)

    plsc.subcore_barrier()
    pltpu.sync_copy(ones_v, counts_sh.at[idx_v], add=True)
    plsc.subcore_barrier()

    pltpu.sync_copy(counts_sh.at[pl.ds(sid * HIST_SLICE, HIST_SLICE)],
                    zeros_v.at[pl.ds(0, HIST_SLICE)])
    pltpu.sync_copy(zeros_v.at[pl.ds(0, HIST_SLICE)],
                    out_hbm.at[pl.ds(cid * HIST_PAD + sid * HIST_SLICE,
                                     HIST_SLICE)])


@jax.jit
def _sc_hist(src, dst, ntype_dict):
    mesh = plsc.VectorSubcoreMesh(core_axis_name="c", subcore_axis_name="s")
    k = functools.partial(
        pl.kernel,
        mesh=mesh,
        out_type=jax.ShapeDtypeStruct((NC * HIST_PAD,), jnp.float32),
        scratch_types=[
            pltpu.VMEM((N,), jnp.int32),
            pltpu.VMEM((EDGES_PER_W,), jnp.int32),
            pltpu.VMEM((EDGES_PER_W,), jnp.int32),
            pltpu.VMEM((EVENTS_PER_W,), jnp.int32),
            pltpu.VMEM((EVENTS_PER_W,), jnp.float32),
            pltpu.VMEM((ZBUF,), jnp.float32),
            pltpu.VMEM_SHARED((HIST,), jnp.float32),
            pltpu.SemaphoreType.DMA,
        ],
        compiler_params=pltpu.CompilerParams(needs_layout_passes=False),
    )(_sc_hist_kernel)
    return k(src, dst, ntype_dict)


def _epilogue_kernel(parts_ref, emb_ref, w_ref, b_ref, out_ref):
    p = parts_ref[...]
    c2 = p[0:ROWS] + p[640:640 + ROWS]
    m = jnp.dot(emb_ref[...], w_ref[...].T,
                preferred_element_type=jnp.float32)
    mt = jnp.concatenate([m] * 16, axis=0)
    mt = jnp.concatenate([mt] * 16, axis=1)
    ii = lax.broadcasted_iota(jnp.int32, (128, 128), 0) // T
    jj = lax.broadcasted_iota(jnp.int32, (128, 128), 1) // T
    blk = ii == jj
    bdm = jnp.where(blk, mt, 0.0)
    dmask = jnp.where(blk, 1.0, 0.0)
    deg = jnp.maximum(jnp.dot(c2, dmask, preferred_element_type=jnp.float32),
                      1.0)
    bw = jnp.concatenate([b_ref[...]] * 16, axis=1)
    out_ref[...] = (jnp.dot(c2, bdm, preferred_element_type=jnp.float32) / deg
                    + bw)


@jax.jit
def _epilogue(parts, embeddings, W, b):
    return pl.pallas_call(
        _epilogue_kernel,
        out_shape=jax.ShapeDtypeStruct((ROWS, 128), jnp.float32),
    )(parts, embeddings, W, b.reshape(1, T))


def kernel(src, dst, ntype_dict, embeddings, W, b):
    src = src.astype(jnp.int32)
    dst = dst.astype(jnp.int32)
    parts = _sc_hist(src, dst, ntype_dict.astype(jnp.int32))
    out = _epilogue(parts.reshape(NC * HIST_PAD // 128, 128),
                    embeddings, W, b)
    return out.reshape(N, T)


# speedup vs baseline: 154.8873x; 1.0576x over previous
import functools

import jax
import jax.numpy as jnp
from jax import lax
from jax.experimental import pallas as pl
from jax.experimental.pallas import tpu as pltpu
from jax.experimental.pallas import tpu_sc as plsc

N = 10000
E = 320000
D = 128
T = 8

NC = 2
NS = 16
NW = NC * NS
EDGES_PER_W = E // NW
EVENTS_PER_W = 2 * EDGES_PER_W
HIST = N * T
HIST_PAD = 81920
ROWS = HIST // 128
HIST_SLICE = HIST // NS
ZBUF = HIST_SLICE + 8


def _sc_hist_kernel(src_hbm, dst_hbm, ntype_hbm, out_hbm,
                    ntype_v, src_v, dst_v, idx_v, ones_v, zeros_v,
                    counts_sh, sem):
    cid = lax.axis_index("c")
    sid = lax.axis_index("s")
    wid = sid * NC + cid
    base = wid * EDGES_PER_W

    cp_nt = pltpu.async_copy(ntype_hbm, ntype_v, sem)
    cp_src = pltpu.async_copy(src_hbm.at[pl.ds(base, EDGES_PER_W)],
                              src_v.at[pl.ds(0, EDGES_PER_W)], sem)
    cp_dst = pltpu.async_copy(dst_hbm.at[pl.ds(base, EDGES_PER_W)],
                              dst_v.at[pl.ds(0, EDGES_PER_W)], sem)

    one16 = jnp.full((16,), 1.0, jnp.float32)
    zero16 = jnp.zeros((16,), jnp.float32)

    def fill_ones(i, _):
        ones_v[pl.ds(i * 16, 16)] = one16
        return 0

    lax.fori_loop(0, EVENTS_PER_W // 16, fill_ones, 0, unroll=8)

    def fill_zeros(i, _):
        zeros_v[pl.ds(i * 16, 16)] = zero16
        return 0

    lax.fori_loop(0, ZBUF // 16, fill_zeros, 0, unroll=8)

    pltpu.sync_copy(zeros_v.at[pl.ds(0, HIST_SLICE)],
                    counts_sh.at[pl.ds(sid * HIST_SLICE, HIST_SLICE)])
    @pl.when(sid == 0)
    def _():
        pltpu.sync_copy(zeros_v.at[pl.ds(0, HIST_PAD - HIST)],
                        out_hbm.at[pl.ds(cid * HIST_PAD + HIST,
                                         HIST_PAD - HIST)])

    cp_nt.wait()
    cp_src.wait()
    cp_dst.wait()

    nblk = EDGES_PER_W // 16

    def _load(i):
        return src_v[pl.ds(i * 16, 16)], dst_v[pl.ds(i * 16, 16)]

    def _gather(s, d):
        return (plsc.load_gather(ntype_v, [s]), plsc.load_gather(ntype_v, [d]))

    def _store(i, s, d, ts, td):
        off = i * 16
        idx_v[pl.ds(off, 16)] = d * 8 + ts
        idx_v[pl.ds(EDGES_PER_W + off, 16)] = s * 8 + td

    s0, d0 = _load(0)
    s1, d1 = _load(1)
    ts0, td0 = _gather(s0, d0)

    def body(i, carry):
        sn, dn, sc, dc, tsc, tdc = carry
        _store(i, sc, dc, tsc, tdc)
        tsn, tdn = _gather(sn, dn)
        s2, d2 = _load(i + 2)
        return (s2, d2, sn, dn, tsn, tdn)

    carry = lax.fori_loop(0, nblk - 1, body, (s1, d1, s0, d0, ts0, td0),
                          unroll=4)
    _, _, sl, dl, tsl, tdl = carry
    _store(nblk - 1, sl, dl, tsl, tdl)

    plsc.subcore_barrier()
    pltpu.sync_copy(ones_v, counts_sh.at[idx_v], add=True)
    plsc.subcore_barrier()

    pltpu.sync_copy(counts_sh.at[pl.ds(sid * HIST_SLICE, HIST_SLICE)],
                    zeros_v.at[pl.ds(0, HIST_SLICE)])
    pltpu.sync_copy(zeros_v.at[pl.ds(0, HIST_SLICE)],
                    out_hbm.at[pl.ds(cid * HIST_PAD + sid * HIST_SLICE,
                                     HIST_SLICE)])


@jax.jit
def _sc_hist(src, dst, ntype_dict):
    mesh = plsc.VectorSubcoreMesh(core_axis_name="c", subcore_axis_name="s")
    k = functools.partial(
        pl.kernel,
        mesh=mesh,
        out_type=jax.ShapeDtypeStruct((NC * HIST_PAD,), jnp.float32),
        scratch_types=[
            pltpu.VMEM((N,), jnp.int32),
            pltpu.VMEM((EDGES_PER_W + 16,), jnp.int32),
            pltpu.VMEM((EDGES_PER_W + 16,), jnp.int32),
            pltpu.VMEM((EVENTS_PER_W,), jnp.int32),
            pltpu.VMEM((EVENTS_PER_W,), jnp.float32),
            pltpu.VMEM((ZBUF,), jnp.float32),
            pltpu.VMEM_SHARED((HIST,), jnp.float32),
            pltpu.SemaphoreType.DMA,
        ],
        compiler_params=pltpu.CompilerParams(needs_layout_passes=False),
    )(_sc_hist_kernel)
    return k(src, dst, ntype_dict)


def _epilogue_kernel(parts_ref, emb_ref, w_ref, b_ref, out_ref):
    p = parts_ref[...]
    c2 = p[0:ROWS] + p[640:640 + ROWS]
    m = jnp.dot(emb_ref[...], w_ref[...].T,
                preferred_element_type=jnp.float32)
    mt = jnp.concatenate([m] * 16, axis=0)
    mt = jnp.concatenate([mt] * 16, axis=1)
    ii = lax.broadcasted_iota(jnp.int32, (128, 128), 0) // T
    jj = lax.broadcasted_iota(jnp.int32, (128, 128), 1) // T
    blk = ii == jj
    bdm = jnp.where(blk, mt, 0.0)
    dmask = jnp.where(blk, 1.0, 0.0)
    deg = jnp.maximum(jnp.dot(c2, dmask, preferred_element_type=jnp.float32),
                      1.0)
    bw = jnp.concatenate([b_ref[...]] * 16, axis=1)
    out_ref[...] = (jnp.dot(c2, bdm, preferred_element_type=jnp.float32) / deg
                    + bw)


@jax.jit
def _epilogue(parts, embeddings, W, b):
    return pl.pallas_call(
        _epilogue_kernel,
        out_shape=jax.ShapeDtypeStruct((ROWS, 128), jnp.float32),
    )(parts, embeddings, W, b.reshape(1, T))


def kernel(src, dst, ntype_dict, embeddings, W, b):
    src = src.astype(jnp.int32)
    dst = dst.astype(jnp.int32)
    parts = _sc_hist(src, dst, ntype_dict.astype(jnp.int32))
    out = _epilogue(parts.reshape(NC * HIST_PAD // 128, 128),
                    embeddings, W, b)
    return out.reshape(N, T)


# speedup vs baseline: 157.9701x; 1.0199x over previous
import functools

import jax
import jax.numpy as jnp
from jax import lax
from jax.experimental import pallas as pl
from jax.experimental.pallas import tpu as pltpu
from jax.experimental.pallas import tpu_sc as plsc

N = 10000
E = 320000
D = 128
T = 8

NC = 2
NS = 16
NW = NC * NS
EDGES_PER_W = E // NW
EVENTS_PER_W = 2 * EDGES_PER_W
HIST = N * T
HIST_T = 80128
HIST_PAD = 81920
ROWS = HIST // 128
HIST_SLICE = HIST // NS
INIT_SLICE = HIST_T // NS
ZBUF = INIT_SLICE
NBLK = EDGES_PER_W // 16
CBLK = 128
NCHUNK = 5
CPAD = NCHUNK * CBLK


def _sc_hist_kernel(src_hbm, dst_hbm, ntype_hbm, out_hbm,
                    ntype_v, src_v, dst_v, *rest):
    idx_refs = rest[:2 * NCHUNK]
    ones_v, zeros_v, counts_sh, sem, sem2 = rest[2 * NCHUNK:]
    cid = lax.axis_index("c")
    sid = lax.axis_index("s")
    wid = sid * NC + cid
    base = wid * EDGES_PER_W

    cp_nt = pltpu.async_copy(ntype_hbm, ntype_v, sem)
    cp_src = pltpu.async_copy(src_hbm.at[pl.ds(base, EDGES_PER_W)],
                              src_v.at[pl.ds(0, EDGES_PER_W)], sem)
    cp_dst = pltpu.async_copy(dst_hbm.at[pl.ds(base, EDGES_PER_W)],
                              dst_v.at[pl.ds(0, EDGES_PER_W)], sem)

    one16 = jnp.full((16,), 1.0, jnp.float32)
    zero16 = jnp.zeros((16,), jnp.float32)
    trash16 = HIST + lax.iota(jnp.int32, 16)

    def fill_ones(i, _):
        ones_v[pl.ds(i * 16, 16)] = one16
        return 0

    lax.fori_loop(0, CBLK, fill_ones, 0, unroll=8)

    def fill_zeros(i, _):
        zeros_v[pl.ds(i * 16, 16)] = zero16
        return 0

    lax.fori_loop(0, ZBUF // 16, fill_zeros, 0, unroll=8)

    tail0 = (NBLK - (NCHUNK - 1) * CBLK) * 16

    def fill_trash(i, _):
        idx_refs[2 * (NCHUNK - 1)][pl.ds(tail0 + i * 16, 16)] = trash16
        idx_refs[2 * (NCHUNK - 1) + 1][pl.ds(tail0 + i * 16, 16)] = trash16
        return 0

    lax.fori_loop(0, CPAD - NBLK, fill_trash, 0, unroll=4)

    pltpu.sync_copy(zeros_v, counts_sh.at[pl.ds(sid * INIT_SLICE, INIT_SLICE)])
    @pl.when(sid == 0)
    def _():
        pltpu.sync_copy(zeros_v.at[pl.ds(0, HIST_PAD - HIST)],
                        out_hbm.at[pl.ds(cid * HIST_PAD + HIST,
                                         HIST_PAD - HIST)])

    cp_nt.wait()
    cp_src.wait()
    cp_dst.wait()
    plsc.subcore_barrier()

    def _load(i):
        return src_v[pl.ds(i * 16, 16)], dst_v[pl.ds(i * 16, 16)]

    def _gather(s, d):
        return (plsc.load_gather(ntype_v, [s]), plsc.load_gather(ntype_v, [d]))

    scatters = []
    for c in range(NCHUNK):
        blk0 = c * CBLK
        nb = min(CBLK, NBLK - blk0)

        def _store(l, s, d, ts, td, _c=c):
            off = l * 16
            idx_refs[2 * _c][pl.ds(off, 16)] = d * 8 + ts
            idx_refs[2 * _c + 1][pl.ds(off, 16)] = s * 8 + td

        s0, d0 = _load(blk0)
        s1, d1 = _load(blk0 + 1)
        ts0, td0 = _gather(s0, d0)

        def body(l, carry, _blk0=blk0, _store=_store):
            sn, dn, sc, dc, tsc, tdc = carry
            _store(l, sc, dc, tsc, tdc)
            tsn, tdn = _gather(sn, dn)
            s2, d2 = _load(_blk0 + l + 2)
            return (s2, d2, sn, dn, tsn, tdn)

        carry = lax.fori_loop(0, nb - 1, body, (s1, d1, s0, d0, ts0, td0),
                              unroll=4)
        _, _, sl, dl, tsl, tdl = carry
        _store(nb - 1, sl, dl, tsl, tdl)
        scatters.append(
            pltpu.async_copy(ones_v, counts_sh.at[idx_refs[2 * c]],
                             sem2, add=True))
        scatters.append(
            pltpu.async_copy(ones_v, counts_sh.at[idx_refs[2 * c + 1]],
                             sem2, add=True))

    for cp in scatters:
        cp.wait()
    plsc.subcore_barrier()

    pltpu.sync_copy(counts_sh.at[pl.ds(sid * HIST_SLICE, HIST_SLICE)],
                    zeros_v.at[pl.ds(0, HIST_SLICE)])
    pltpu.sync_copy(zeros_v.at[pl.ds(0, HIST_SLICE)],
                    out_hbm.at[pl.ds(cid * HIST_PAD + sid * HIST_SLICE,
                                     HIST_SLICE)])


@jax.jit
def _sc_hist(src, dst, ntype_dict):
    mesh = plsc.VectorSubcoreMesh(core_axis_name="c", subcore_axis_name="s")
    k = functools.partial(
        pl.kernel,
        mesh=mesh,
        out_type=jax.ShapeDtypeStruct((NC * HIST_PAD,), jnp.float32),
        scratch_types=[
            pltpu.VMEM((N,), jnp.int32),
            pltpu.VMEM((EDGES_PER_W + 32,), jnp.int32),
            pltpu.VMEM((EDGES_PER_W + 32,), jnp.int32),
            *[pltpu.VMEM((CBLK * 16,), jnp.int32)
              for _ in range(2 * NCHUNK)],
            pltpu.VMEM((CBLK * 16,), jnp.float32),
            pltpu.VMEM((ZBUF,), jnp.float32),
            pltpu.VMEM_SHARED((HIST_T,), jnp.float32),
            pltpu.SemaphoreType.DMA,
            pltpu.SemaphoreType.DMA,
        ],
        compiler_params=pltpu.CompilerParams(needs_layout_passes=False),
    )(_sc_hist_kernel)
    return k(src, dst, ntype_dict)


def _epilogue_kernel(parts_ref, emb_ref, w_ref, b_ref, out_ref):
    p = parts_ref[...]
    c2 = p[0:ROWS] + p[640:640 + ROWS]
    m = jnp.dot(emb_ref[...], w_ref[...].T,
                preferred_element_type=jnp.float32)
    mt = jnp.concatenate([m] * 16, axis=0)
    mt = jnp.concatenate([mt] * 16, axis=1)
    ii = lax.broadcasted_iota(jnp.int32, (128, 128), 0) // T
    jj = lax.broadcasted_iota(jnp.int32, (128, 128), 1) // T
    blk = ii == jj
    bdm = jnp.where(blk, mt, 0.0)
    dmask = jnp.where(blk, 1.0, 0.0)
    deg = jnp.maximum(jnp.dot(c2, dmask, preferred_element_type=jnp.float32),
                      1.0)
    bw = jnp.concatenate([b_ref[...]] * 16, axis=1)
    out_ref[...] = (jnp.dot(c2, bdm, preferred_element_type=jnp.float32) / deg
                    + bw)


@jax.jit
def _epilogue(parts, embeddings, W, b):
    return pl.pallas_call(
        _epilogue_kernel,
        out_shape=jax.ShapeDtypeStruct((ROWS, 128), jnp.float32),
    )(parts, embeddings, W, b.reshape(1, T))


def kernel(src, dst, ntype_dict, embeddings, W, b):
    src = src.astype(jnp.int32)
    dst = dst.astype(jnp.int32)
    parts = _sc_hist(src, dst, ntype_dict.astype(jnp.int32))
    out = _epilogue(parts.reshape(NC * HIST_PAD // 128, 128),
                    embeddings, W, b)
    return out.reshape(N, T)


# speedup vs baseline: 158.3826x; 1.0026x over previous
import functools

import jax
import jax.numpy as jnp
from jax import lax
from jax.experimental import pallas as pl
from jax.experimental.pallas import tpu as pltpu
from jax.experimental.pallas import tpu_sc as plsc
from jax.experimental import layout as _layout
from jax._src import pjit as _pjit

N = 10000
E = 320000
D = 128
T = 8

NC = 2
NS = 16
NW = NC * NS
EDGES_PER_W = E // NW
EVENTS_PER_W = 2 * EDGES_PER_W
HIST = N * T
HIST_T = 80128
HIST_PAD = 81920
ROWS = HIST // 128
HIST_SLICE = HIST // NS
INIT_SLICE = HIST_T // NS
ZBUF = INIT_SLICE
NBLK = EDGES_PER_W // 16
CBLK = 128
NCHUNK = 5
CPAD = NCHUNK * CBLK


def _sc_hist_kernel(src_hbm, dst_hbm, ntype_hbm, out_hbm,
                    ntype_v, src_v, dst_v, *rest):
    idx_refs = rest[:2 * NCHUNK]
    ones_v, zeros_v, counts_sh, sem, sem2 = rest[2 * NCHUNK:]
    cid = lax.axis_index("c")
    sid = lax.axis_index("s")
    wid = sid * NC + cid
    base = wid * EDGES_PER_W

    cp_nt = pltpu.async_copy(ntype_hbm, ntype_v, sem)
    cp_src = pltpu.async_copy(src_hbm.at[pl.ds(base, EDGES_PER_W)],
                              src_v.at[pl.ds(0, EDGES_PER_W)], sem)
    cp_dst = pltpu.async_copy(dst_hbm.at[pl.ds(base, EDGES_PER_W)],
                              dst_v.at[pl.ds(0, EDGES_PER_W)], sem)

    one16 = jnp.full((16,), 1.0, jnp.float32)
    zero16 = jnp.zeros((16,), jnp.float32)
    trash16 = HIST + lax.iota(jnp.int32, 16)

    def fill_ones(i, _):
        ones_v[pl.ds(i * 16, 16)] = one16
        return 0

    lax.fori_loop(0, CBLK, fill_ones, 0, unroll=8)

    def fill_zeros(i, _):
        zeros_v[pl.ds(i * 16, 16)] = zero16
        return 0

    lax.fori_loop(0, ZBUF // 16, fill_zeros, 0, unroll=8)

    tail0 = (NBLK - (NCHUNK - 1) * CBLK) * 16

    def fill_trash(i, _):
        idx_refs[2 * (NCHUNK - 1)][pl.ds(tail0 + i * 16, 16)] = trash16
        idx_refs[2 * (NCHUNK - 1) + 1][pl.ds(tail0 + i * 16, 16)] = trash16
        return 0

    lax.fori_loop(0, CPAD - NBLK, fill_trash, 0, unroll=4)

    pltpu.sync_copy(zeros_v, counts_sh.at[pl.ds(sid * INIT_SLICE, INIT_SLICE)])
    @pl.when(sid == 0)
    def _():
        pltpu.sync_copy(zeros_v.at[pl.ds(0, HIST_PAD - HIST)],
                        out_hbm.at[pl.ds(cid * HIST_PAD + HIST,
                                         HIST_PAD - HIST)])

    cp_nt.wait()
    cp_src.wait()
    cp_dst.wait()
    plsc.subcore_barrier()

    def _load(i):
        return src_v[pl.ds(i * 16, 16)], dst_v[pl.ds(i * 16, 16)]

    def _gather(s, d):
        return (plsc.load_gather(ntype_v, [s]), plsc.load_gather(ntype_v, [d]))

    scatters = []
    for c in range(NCHUNK):
        blk0 = c * CBLK
        nb = min(CBLK, NBLK - blk0)

        def _store(l, s, d, ts, td, _c=c):
            off = l * 16
            idx_refs[2 * _c][pl.ds(off, 16)] = d * 8 + ts
            idx_refs[2 * _c + 1][pl.ds(off, 16)] = s * 8 + td

        s0, d0 = _load(blk0)
        s1, d1 = _load(blk0 + 1)
        ts0, td0 = _gather(s0, d0)

        def body(l, carry, _blk0=blk0, _store=_store):
            sn, dn, sc, dc, tsc, tdc = carry
            _store(l, sc, dc, tsc, tdc)
            tsn, tdn = _gather(sn, dn)
            s2, d2 = _load(_blk0 + l + 2)
            return (s2, d2, sn, dn, tsn, tdn)

        carry = lax.fori_loop(0, nb - 1, body, (s1, d1, s0, d0, ts0, td0),
                              unroll=4)
        _, _, sl, dl, tsl, tdl = carry
        _store(nb - 1, sl, dl, tsl, tdl)
        scatters.append(
            pltpu.async_copy(ones_v, counts_sh.at[idx_refs[2 * c]],
                             sem2, add=True))
        scatters.append(
            pltpu.async_copy(ones_v, counts_sh.at[idx_refs[2 * c + 1]],
                             sem2, add=True))

    for cp in scatters:
        cp.wait()
    plsc.subcore_barrier()

    pltpu.sync_copy(counts_sh.at[pl.ds(sid * HIST_SLICE, HIST_SLICE)],
                    zeros_v.at[pl.ds(0, HIST_SLICE)])
    pltpu.sync_copy(zeros_v.at[pl.ds(0, HIST_SLICE)],
                    out_hbm.at[pl.ds(cid * HIST_PAD + sid * HIST_SLICE,
                                     HIST_SLICE)])


@jax.jit
def _sc_hist(src, dst, ntype_dict):
    mesh = plsc.VectorSubcoreMesh(core_axis_name="c", subcore_axis_name="s")
    k = functools.partial(
        pl.kernel,
        mesh=mesh,
        out_type=jax.ShapeDtypeStruct((NC * HIST_PAD,), jnp.float32),
        scratch_types=[
            pltpu.VMEM((N,), jnp.int32),
            pltpu.VMEM((EDGES_PER_W + 32,), jnp.int32),
            pltpu.VMEM((EDGES_PER_W + 32,), jnp.int32),
            *[pltpu.VMEM((CBLK * 16,), jnp.int32)
              for _ in range(2 * NCHUNK)],
            pltpu.VMEM((CBLK * 16,), jnp.float32),
            pltpu.VMEM((ZBUF,), jnp.float32),
            pltpu.VMEM_SHARED((HIST_T,), jnp.float32),
            pltpu.SemaphoreType.DMA,
            pltpu.SemaphoreType.DMA,
        ],
        compiler_params=pltpu.CompilerParams(needs_layout_passes=False),
    )(_sc_hist_kernel)
    return k(src, dst, ntype_dict)


def _epilogue_kernel(parts_ref, emb_ref, w_ref, b_ref, out_ref):
    p = parts_ref[...]
    c2 = p[0:ROWS] + p[640:640 + ROWS]
    m = jnp.dot(emb_ref[...], w_ref[...].T,
                preferred_element_type=jnp.float32)
    mt = jnp.concatenate([m] * 16, axis=0)
    mt = jnp.concatenate([mt] * 16, axis=1)
    ii = lax.broadcasted_iota(jnp.int32, (128, 128), 0) // T
    jj = lax.broadcasted_iota(jnp.int32, (128, 128), 1) // T
    blk = ii == jj
    bdm = jnp.where(blk, mt, 0.0)
    dmask = jnp.where(blk, 1.0, 0.0)
    deg = jnp.maximum(jnp.dot(c2, dmask, preferred_element_type=jnp.float32),
                      1.0)
    bw = jnp.concatenate([b_ref[...]] * 16, axis=1)
    out_ref[...] = (jnp.dot(c2, bdm, preferred_element_type=jnp.float32) / deg
                    + bw)


@jax.jit
def _epilogue(parts, embeddings, W, b):
    return pl.pallas_call(
        _epilogue_kernel,
        out_shape=jax.ShapeDtypeStruct((ROWS, 128), jnp.float32),
    )(parts, embeddings, W, b.reshape(1, T))


def kernel(src, dst, ntype_dict, embeddings, W, b):
    src = src.astype(jnp.int32)
    dst = dst.astype(jnp.int32)
    parts = _sc_hist(src, dst, ntype_dict.astype(jnp.int32))
    out = _epilogue(parts.reshape(NC * HIST_PAD // 128, 128),
                    embeddings, W, b)
    out = out.reshape(N, T)
    return _pjit.with_layout_constraint(
        out, _layout.Layout(major_to_minor=(1, 0), tiling=((8, 128),)))


# speedup vs baseline: 158.5911x; 1.0013x over previous
import functools

import jax
import jax.numpy as jnp
from jax import lax
from jax.experimental import pallas as pl
from jax.experimental.pallas import tpu as pltpu
from jax.experimental.pallas import tpu_sc as plsc

N = 10000
E = 320000
D = 128
T = 8

NC = 2
NS = 16
NW = NC * NS
EDGES_PER_W = E // NW
EVENTS_PER_W = 2 * EDGES_PER_W
HIST = N * T
HIST_T = 80128
HIST_PAD = 81920
ROWS = HIST // 128
HIST_SLICE = HIST // NS
INIT_SLICE = HIST_T // NS
ZBUF = INIT_SLICE
NBLK = EDGES_PER_W // 16
CBLK = 128
NCHUNK = 5
CPAD = NCHUNK * CBLK


def _sc_hist_kernel(src_hbm, dst_hbm, ntype_hbm, out_hbm,
                    ntype_v, src_v, dst_v, *rest):
    idx_refs = rest[:2 * NCHUNK]
    ones_v, zeros_v, counts_sh, sem, sem2 = rest[2 * NCHUNK:]
    cid = lax.axis_index("c")
    sid = lax.axis_index("s")
    wid = sid * NC + cid
    base = wid * EDGES_PER_W

    cp_nt = pltpu.async_copy(ntype_hbm, ntype_v, sem)
    cp_src = pltpu.async_copy(src_hbm.at[pl.ds(base, EDGES_PER_W)],
                              src_v.at[pl.ds(0, EDGES_PER_W)], sem)
    cp_dst = pltpu.async_copy(dst_hbm.at[pl.ds(base, EDGES_PER_W)],
                              dst_v.at[pl.ds(0, EDGES_PER_W)], sem)

    one16 = jnp.full((16,), 1.0, jnp.float32)
    zero16 = jnp.zeros((16,), jnp.float32)
    trash16 = HIST + lax.iota(jnp.int32, 16)

    def fill_ones(i, _):
        ones_v[pl.ds(i * 16, 16)] = one16
        return 0

    lax.fori_loop(0, CBLK, fill_ones, 0, unroll=8)

    def fill_zeros(i, _):
        zeros_v[pl.ds(i * 16, 16)] = zero16
        return 0

    lax.fori_loop(0, ZBUF // 16, fill_zeros, 0, unroll=8)

    tail0 = (NBLK - (NCHUNK - 1) * CBLK) * 16

    def fill_trash(i, _):
        idx_refs[2 * (NCHUNK - 1)][pl.ds(tail0 + i * 16, 16)] = trash16
        idx_refs[2 * (NCHUNK - 1) + 1][pl.ds(tail0 + i * 16, 16)] = trash16
        return 0

    lax.fori_loop(0, CPAD - NBLK, fill_trash, 0, unroll=4)

    pltpu.sync_copy(zeros_v, counts_sh.at[pl.ds(sid * INIT_SLICE, INIT_SLICE)])
    @pl.when(sid == 0)
    def _():
        pltpu.sync_copy(zeros_v.at[pl.ds(0, HIST_PAD - HIST)],
                        out_hbm.at[pl.ds(cid * HIST_PAD + HIST,
                                         HIST_PAD - HIST)])

    cp_nt.wait()
    cp_src.wait()
    cp_dst.wait()
    plsc.subcore_barrier()

    def _load(i):
        return src_v[pl.ds(i * 16, 16)], dst_v[pl.ds(i * 16, 16)]

    def _gather(s, d):
        return (plsc.load_gather(ntype_v, [s]), plsc.load_gather(ntype_v, [d]))

    scatters = []
    for c in range(NCHUNK):
        blk0 = c * CBLK
        nb = min(CBLK, NBLK - blk0)

        def _store(l, s, d, ts, td, _c=c):
            off = l * 16
            idx_refs[2 * _c][pl.ds(off, 16)] = d * 8 + ts
            idx_refs[2 * _c + 1][pl.ds(off, 16)] = s * 8 + td

        s0, d0 = _load(blk0)
        s1, d1 = _load(blk0 + 1)
        ts0, td0 = _gather(s0, d0)

        def body(l, carry, _blk0=blk0, _store=_store):
            sn, dn, sc, dc, tsc, tdc = carry
            _store(l, sc, dc, tsc, tdc)
            tsn, tdn = _gather(sn, dn)
            s2, d2 = _load(_blk0 + l + 2)
            return (s2, d2, sn, dn, tsn, tdn)

        carry = lax.fori_loop(0, nb - 1, body, (s1, d1, s0, d0, ts0, td0),
                              unroll=4)
        _, _, sl, dl, tsl, tdl = carry
        _store(nb - 1, sl, dl, tsl, tdl)
        scatters.append(
            pltpu.async_copy(ones_v, counts_sh.at[idx_refs[2 * c]],
                             sem2, add=True))
        scatters.append(
            pltpu.async_copy(ones_v, counts_sh.at[idx_refs[2 * c + 1]],
                             sem2, add=True))

    for cp in scatters:
        cp.wait()
    plsc.subcore_barrier()

    pltpu.sync_copy(counts_sh.at[pl.ds(sid * HIST_SLICE, HIST_SLICE)],
                    zeros_v.at[pl.ds(0, HIST_SLICE)])
    pltpu.sync_copy(zeros_v.at[pl.ds(0, HIST_SLICE)],
                    out_hbm.at[pl.ds(cid * HIST_PAD + sid * HIST_SLICE,
                                     HIST_SLICE)])


@jax.jit
def _sc_hist(src, dst, ntype_dict):
    mesh = plsc.VectorSubcoreMesh(core_axis_name="c", subcore_axis_name="s")
    k = functools.partial(
        pl.kernel,
        mesh=mesh,
        out_type=jax.ShapeDtypeStruct((NC * HIST_PAD,), jnp.float32),
        scratch_types=[
            pltpu.VMEM((N,), jnp.int32),
            pltpu.VMEM((EDGES_PER_W + 32,), jnp.int32),
            pltpu.VMEM((EDGES_PER_W + 32,), jnp.int32),
            *[pltpu.VMEM((CBLK * 16,), jnp.int32)
              for _ in range(2 * NCHUNK)],
            pltpu.VMEM((CBLK * 16,), jnp.float32),
            pltpu.VMEM((ZBUF,), jnp.float32),
            pltpu.VMEM_SHARED((HIST_T,), jnp.float32),
            pltpu.SemaphoreType.DMA,
            pltpu.SemaphoreType.DMA,
        ],
        compiler_params=pltpu.CompilerParams(needs_layout_passes=False),
    )(_sc_hist_kernel)
    return k(src, dst, ntype_dict)


def _epilogue_kernel(parts_ref, emb_ref, w_ref, b_ref, out_ref):
    p = parts_ref[...]
    c2 = p[0:ROWS] + p[640:640 + ROWS]
    m = jnp.dot(emb_ref[...], w_ref[...].T,
                preferred_element_type=jnp.float32)
    mt = jnp.concatenate([m] * 16, axis=0)
    mt = jnp.concatenate([mt] * 16, axis=1)
    ii = lax.broadcasted_iota(jnp.int32, (128, 128), 0) // T
    jj = lax.broadcasted_iota(jnp.int32, (128, 128), 1) // T
    blk = ii == jj
    bdm = jnp.where(blk, mt, 0.0)
    dmask = jnp.where(blk, 1.0, 0.0)
    deg = jnp.maximum(jnp.dot(c2, dmask, preferred_element_type=jnp.float32),
                      1.0)
    bw = jnp.concatenate([b_ref[...]] * 16, axis=1)
    out_ref[...] = (jnp.dot(c2, bdm, preferred_element_type=jnp.float32) / deg
                    + bw)


@jax.jit
def _epilogue(parts, embeddings, W, b):
    return pl.pallas_call(
        _epilogue_kernel,
        out_shape=jax.ShapeDtypeStruct((ROWS, 128), jnp.float32),
    )(parts, embeddings, W, b.reshape(1, T))


def kernel(src, dst, ntype_dict, embeddings, W, b):
    src = src.astype(jnp.int32)
    dst = dst.astype(jnp.int32)
    parts = _sc_hist(src, dst, ntype_dict.astype(jnp.int32))
    out = _epilogue(parts.reshape(NC * HIST_PAD // 128, 128),
                    embeddings, W, b)
    return out.reshape(N, T)
